# baseline (speedup 1.0000x reference)
import jax
import jax.numpy as jnp
from jax.experimental import pallas as pl
from jax.experimental.pallas import tpu as pltpu


def _body(idx_ref, cqt_ref, pos_ref, out_ref):
    b = pl.program_id(0)
    c = pl.program_id(1)
    sel = idx_ref[b] == c

    @pl.when(sel)
    def _():
        out_ref[...] = pos_ref[...]

    @pl.when(jnp.logical_not(sel))
    def _():
        out_ref[...] = cqt_ref[...]


def kernel(cqt, pos_cqt, channel_idx):
    B, C, F, T = cqt.shape
    idx = channel_idx.astype(jnp.int32)

    grid_spec = pltpu.PrefetchScalarGridSpec(
        num_scalar_prefetch=1,
        grid=(B, C),
        in_specs=[
            pl.BlockSpec((1, 1, F, T), lambda b, c, idx_ref: (b, c, 0, 0)),
            pl.BlockSpec((1, 1, F, T), lambda b, c, idx_ref: (b, idx_ref[b], 0, 0)),
        ],
        out_specs=pl.BlockSpec((1, 1, F, T), lambda b, c, idx_ref: (b, c, 0, 0)),
    )
    return pl.pallas_call(
        _body,
        grid_spec=grid_spec,
        out_shape=jax.ShapeDtypeStruct(cqt.shape, cqt.dtype),
    )(idx, cqt, pos_cqt)


# speedup vs baseline: 1.4882x; 1.4882x over previous
import jax
import jax.numpy as jnp
from jax.experimental import pallas as pl
from jax.experimental.pallas import tpu as pltpu


def _body(idx_ref, cqt_ref, pos_ref, out_ref):
    b = pl.program_id(0)
    out_ref[...] = cqt_ref[...]
    c = idx_ref[b]
    out_ref[0, pl.ds(c, 1), :, :] = pos_ref[0]


def kernel(cqt, pos_cqt, channel_idx):
    B, C, F, T = cqt.shape
    idx = channel_idx.astype(jnp.int32)

    grid_spec = pltpu.PrefetchScalarGridSpec(
        num_scalar_prefetch=1,
        grid=(B,),
        in_specs=[
            pl.BlockSpec((1, C, F, T), lambda b, idx_ref: (b, 0, 0, 0)),
            pl.BlockSpec((1, 1, F, T), lambda b, idx_ref: (b, idx_ref[b], 0, 0)),
        ],
        out_specs=pl.BlockSpec((1, C, F, T), lambda b, idx_ref: (b, 0, 0, 0)),
    )
    return pl.pallas_call(
        _body,
        grid_spec=grid_spec,
        out_shape=jax.ShapeDtypeStruct(cqt.shape, cqt.dtype),
    )(idx, cqt, pos_cqt)


# speedup vs baseline: 1.5988x; 1.0743x over previous
import jax
import jax.numpy as jnp
from jax.experimental import pallas as pl
from jax.experimental.pallas import tpu as pltpu


_BB = 8


def _body(idx_ref, cqt_ref, pos_ref, out_ref):
    b0 = pl.program_id(0) * _BB
    idx_b = jnp.stack([idx_ref[b0 + i] for i in range(_BB)])
    idx_b = idx_b.reshape(_BB, 1, 1, 1)
    c_iota = jax.lax.broadcasted_iota(jnp.int32, (1, cqt_ref.shape[1], 1, 1), 1)
    mask = c_iota == idx_b
    out_ref[...] = jnp.where(mask, pos_ref[...], cqt_ref[...])


def kernel(cqt, pos_cqt, channel_idx):
    B, C, F, T = cqt.shape
    idx = channel_idx.astype(jnp.int32)

    grid_spec = pltpu.PrefetchScalarGridSpec(
        num_scalar_prefetch=1,
        grid=(B // _BB,),
        in_specs=[
            pl.BlockSpec((_BB, C, F, T), lambda b, idx_ref: (b, 0, 0, 0)),
            pl.BlockSpec((_BB, C, F, T), lambda b, idx_ref: (b, 0, 0, 0)),
        ],
        out_specs=pl.BlockSpec((_BB, C, F, T), lambda b, idx_ref: (b, 0, 0, 0)),
    )
    return pl.pallas_call(
        _body,
        grid_spec=grid_spec,
        out_shape=jax.ShapeDtypeStruct(cqt.shape, cqt.dtype),
    )(idx, cqt, pos_cqt)
